# baseline (speedup 1.0000x reference)
import functools

import jax
import jax.numpy as jnp
from jax import lax
from jax.experimental import pallas as pl
from jax.experimental.pallas import tpu as pltpu
from jax.experimental.pallas import tpu_sc as plsc

N = 10000
D = 128
E = 320000

NC = 2
NS = 16
CHUNK = 80
NCHUNK = 125
EPW = NCHUNK * CHUNK
EPAD = NC * NS * EPW
NIDX = 6
NRB = 3
NDUMMY = 16
ROW_STEP = 624


def _sc_scatter_partials(feat, src_p, dst_p):
    mesh = plsc.VectorSubcoreMesh(core_axis_name="c", subcore_axis_name="s")

    @functools.partial(
        pl.kernel,
        out_type=jax.ShapeDtypeStruct((NC * N, D), jnp.float32),
        mesh=mesh,
        scratch_types=[
            pltpu.VMEM_SHARED((N + NDUMMY, D), jnp.float32),
            pltpu.VMEM((NRB, CHUNK, D), jnp.float32),
            *([pltpu.VMEM((CHUNK,), jnp.int32)] * NIDX),
            *([pltpu.VMEM((CHUNK,), jnp.int32)] * NIDX),
            *([pltpu.SemaphoreType.DMA] * NIDX),
            *([pltpu.SemaphoreType.DMA] * NIDX),
            *([pltpu.SemaphoreType.DMA] * NRB),
            *([pltpu.SemaphoreType.DMA] * NRB),
        ],
    )
    def k(feat_hbm, src_hbm, dst_hbm, out_hbm, acc, rows, *rest):
        srci = list(rest[0:NIDX])
        dsti = list(rest[NIDX:2 * NIDX])
        ssem = list(rest[2 * NIDX:3 * NIDX])
        dsem = list(rest[3 * NIDX:4 * NIDX])
        gsem = list(rest[4 * NIDX:4 * NIDX + NRB])
        csem = list(rest[4 * NIDX + NRB:4 * NIDX + 2 * NRB])
        c = lax.axis_index("c")
        s = lax.axis_index("s")
        w = c * NS + s
        base = w * EPW
        row0 = s * ROW_STEP

        def idx_start(m, sl):
            off = base + m * CHUNK
            d1 = pltpu.async_copy(src_hbm.at[pl.ds(off, CHUNK)], srci[sl], ssem[sl])
            d2 = pltpu.async_copy(dst_hbm.at[pl.ds(off, CHUNK)], dsti[sl], dsem[sl])
            return (d1, d2)

        def gather_start(sl, b):
            return pltpu.async_copy(feat_hbm.at[srci[sl]], rows.at[b], gsem[b])

        idesc = [None] * NIDX
        for m0 in range(3):
            idesc[m0] = idx_start(m0, m0)

        zv = jnp.zeros((16,), jnp.float32)

        def zb(i, carry):
            rows[0, i // (D // 16), pl.ds((i % (D // 16)) * 16, 16)] = zv
            return carry

        lax.fori_loop(0, CHUNK * (D // 16), zb, 0)

        def init_to(nrows):
            for j in range(nrows // CHUNK):
                pltpu.sync_copy(rows.at[0], acc.at[pl.ds(row0 + j * CHUNK, CHUNK)])
            rem = nrows % CHUNK
            if rem:
                pltpu.sync_copy(rows.at[0].at[pl.ds(0, rem)],
                                acc.at[pl.ds(row0 + (nrows // CHUNK) * CHUNK, rem)])

        @pl.when((s == NS - 1) & (c == 0))
        def _():
            pltpu.sync_copy(feat_hbm.at[pl.ds(row0, 640)], acc.at[pl.ds(row0, 640)])

        @pl.when((s != NS - 1) & (c == 0))
        def _():
            pltpu.sync_copy(feat_hbm.at[pl.ds(row0, ROW_STEP)],
                            acc.at[pl.ds(row0, ROW_STEP)])

        @pl.when((s == NS - 1) & (c == 1))
        def _():
            init_to(640)

        @pl.when((s != NS - 1) & (c == 1))
        def _():
            init_to(ROW_STEP)

        plsc.subcore_barrier()

        gdesc = [None] * NRB
        sdesc = [None] * NRB
        idesc[0][0].wait()
        gdesc[0] = gather_start(0, 0)
        for m in range(NCHUNK):
            b = m % NRB
            sl = m % NIDX
            if m + 3 < NCHUNK:
                idesc[(m + 3) % NIDX] = idx_start(m + 3, (m + 3) % NIDX)
            if m + 1 < NCHUNK:
                if m >= 2:
                    sdesc[(m + 1) % NRB].wait()
                idesc[(m + 1) % NIDX][0].wait()
                gdesc[(m + 1) % NRB] = gather_start((m + 1) % NIDX, (m + 1) % NRB)
            gdesc[b].wait()
            idesc[sl][1].wait()
            sdesc[b] = pltpu.async_copy(rows.at[b], acc.at[dsti[sl]], csem[b],
                                        add=True)
        for j in range(NCHUNK - 3, NCHUNK):
            sdesc[j % NRB].wait()

        plsc.subcore_barrier()

        @pl.when(s == NS - 1)
        def _():
            pltpu.sync_copy(acc.at[pl.ds(row0, 640)],
                            out_hbm.at[pl.ds(c * N + row0, 640)])

        @pl.when(s != NS - 1)
        def _():
            pltpu.sync_copy(acc.at[pl.ds(row0, ROW_STEP)],
                            out_hbm.at[pl.ds(c * N + row0, ROW_STEP)])

    return k(feat, src_p, dst_p)


def _mlp(partials, W1, b1, W2, b2, final_relu):
    R = 5000
    nblk = N // R

    def body(p0_ref, p1_ref, w1_ref, b1_ref, w2_ref, b2_ref, o_ref):
        h = p0_ref[...] + p1_ref[...]
        h = jnp.dot(h, w1_ref[...], preferred_element_type=jnp.float32) + b1_ref[...]
        h = jnp.maximum(h, 0.0)
        o = jnp.dot(h, w2_ref[...], preferred_element_type=jnp.float32) + b2_ref[...]
        if final_relu:
            o = jnp.maximum(o, 0.0)
        o_ref[...] = o

    return pl.pallas_call(
        body,
        grid=(nblk,),
        in_specs=[
            pl.BlockSpec((R, D), lambda i: (i, 0)),
            pl.BlockSpec((R, D), lambda i: (i + nblk, 0)),
            pl.BlockSpec((D, D), lambda i: (0, 0)),
            pl.BlockSpec((1, D), lambda i: (0, 0)),
            pl.BlockSpec((D, D), lambda i: (0, 0)),
            pl.BlockSpec((1, D), lambda i: (0, 0)),
        ],
        out_specs=pl.BlockSpec((R, D), lambda i: (i, 0)),
        out_shape=jax.ShapeDtypeStruct((N, D), jnp.float32),
    )(partials, partials, W1, b1.reshape(1, D), W2, b2.reshape(1, D))


def kernel(x, edge_index, W1_0, b1_0, W2_0, b2_0, W1_1, b1_1, W2_1, b2_1):
    src = edge_index[0].astype(jnp.int32)
    dst = edge_index[1].astype(jnp.int32)
    pad = EPAD - E
    if pad:
        src_p = jnp.concatenate([src, jnp.zeros((pad,), jnp.int32)])
        dst_p = jnp.concatenate([dst, jnp.full((pad,), N, jnp.int32)])
    else:
        src_p, dst_p = src, dst
    p = _sc_scatter_partials(x, src_p, dst_p)
    h = _mlp(p, W1_0, b1_0, W2_0, b2_0, final_relu=True)
    p = _sc_scatter_partials(h, src_p, dst_p)
    return _mlp(p, W1_1, b1_1, W2_1, b2_1, final_relu=False)


# speedup vs baseline: 1.0128x; 1.0128x over previous
import functools

import jax
import jax.numpy as jnp
from jax import lax
from jax.experimental import pallas as pl
from jax.experimental.pallas import tpu as pltpu
from jax.experimental.pallas import tpu_sc as plsc

N = 10000
D = 128
E = 320000

NC = 2
NS = 16
CHUNK = 80
NCHUNK = 125
EPW = NCHUNK * CHUNK
EPAD = NC * NS * EPW
NIDX = 6
NRB = 3
NDUMMY = 16
ROW_STEP = 624


def _sc_scatter_partials(feat, src_p, dst_p):
    mesh = plsc.VectorSubcoreMesh(core_axis_name="c", subcore_axis_name="s")

    @functools.partial(
        pl.kernel,
        out_type=jax.ShapeDtypeStruct((NC * N, D), jnp.float32),
        mesh=mesh,
        scratch_types=[
            pltpu.VMEM_SHARED((N + NDUMMY, D), jnp.float32),
            pltpu.VMEM((NRB, CHUNK, D), jnp.float32),
            *([pltpu.VMEM((CHUNK,), jnp.int32)] * NIDX),
            *([pltpu.VMEM((CHUNK,), jnp.int32)] * NIDX),
            *([pltpu.SemaphoreType.DMA] * NIDX),
            *([pltpu.SemaphoreType.DMA] * NIDX),
            *([pltpu.SemaphoreType.DMA] * NRB),
            *([pltpu.SemaphoreType.DMA] * NRB),
        ],
    )
    def k(feat_hbm, src_hbm, dst_hbm, out_hbm, acc, rows, *rest):
        srci = list(rest[0:NIDX])
        dsti = list(rest[NIDX:2 * NIDX])
        ssem = list(rest[2 * NIDX:3 * NIDX])
        dsem = list(rest[3 * NIDX:4 * NIDX])
        gsem = list(rest[4 * NIDX:4 * NIDX + NRB])
        csem = list(rest[4 * NIDX + NRB:4 * NIDX + 2 * NRB])
        c = lax.axis_index("c")
        s = lax.axis_index("s")
        w = c * NS + s
        base = w * EPW
        row0 = s * ROW_STEP

        def idx_start(m, sl):
            off = base + m * CHUNK
            d1 = pltpu.async_copy(src_hbm.at[pl.ds(off, CHUNK)], srci[sl], ssem[sl])
            d2 = pltpu.async_copy(dst_hbm.at[pl.ds(off, CHUNK)], dsti[sl], dsem[sl])
            return (d1, d2)

        def gather_start(sl, b):
            return pltpu.async_copy(feat_hbm.at[srci[sl]], rows.at[b], gsem[b])

        idesc = [None] * NIDX
        for m0 in range(3):
            idesc[m0] = idx_start(m0, m0)

        zv = jnp.zeros((16,), jnp.float32)

        def zb(i, carry):
            rows[0, i // (D // 16), pl.ds((i % (D // 16)) * 16, 16)] = zv
            return carry

        lax.fori_loop(0, CHUNK * (D // 16), zb, 0)

        def init_to(nrows):
            for j in range(nrows // CHUNK):
                pltpu.sync_copy(rows.at[0], acc.at[pl.ds(row0 + j * CHUNK, CHUNK)])
            rem = nrows % CHUNK
            if rem:
                pltpu.sync_copy(rows.at[0].at[pl.ds(0, rem)],
                                acc.at[pl.ds(row0 + (nrows // CHUNK) * CHUNK, rem)])

        @pl.when(s == NS - 1)
        def _():
            init_to(640)

        @pl.when(s != NS - 1)
        def _():
            init_to(ROW_STEP)

        plsc.subcore_barrier()

        gdesc = [None] * NRB
        sdesc = [None] * NRB
        idesc[0][0].wait()
        gdesc[0] = gather_start(0, 0)
        for m in range(NCHUNK):
            b = m % NRB
            sl = m % NIDX
            if m + 3 < NCHUNK:
                idesc[(m + 3) % NIDX] = idx_start(m + 3, (m + 3) % NIDX)
            if m + 1 < NCHUNK:
                if m >= 2:
                    sdesc[(m + 1) % NRB].wait()
                idesc[(m + 1) % NIDX][0].wait()
                gdesc[(m + 1) % NRB] = gather_start((m + 1) % NIDX, (m + 1) % NRB)
            gdesc[b].wait()
            idesc[sl][1].wait()
            sdesc[b] = pltpu.async_copy(rows.at[b], acc.at[dsti[sl]], csem[b],
                                        add=True)
        for j in range(NCHUNK - 3, NCHUNK):
            sdesc[j % NRB].wait()

        plsc.subcore_barrier()

        @pl.when(s == NS - 1)
        def _():
            pltpu.sync_copy(acc.at[pl.ds(row0, 640)],
                            out_hbm.at[pl.ds(c * N + row0, 640)])

        @pl.when(s != NS - 1)
        def _():
            pltpu.sync_copy(acc.at[pl.ds(row0, ROW_STEP)],
                            out_hbm.at[pl.ds(c * N + row0, ROW_STEP)])

    return k(feat, src_p, dst_p)


def _mlp(xin, partials, W1, b1, W2, b2, final_relu):
    R = 5000
    nblk = N // R

    def body(x_ref, p0_ref, p1_ref, w1_ref, b1_ref, w2_ref, b2_ref, o_ref):
        h = x_ref[...] + p0_ref[...] + p1_ref[...]
        h = jnp.dot(h.astype(jnp.bfloat16), w1_ref[...].astype(jnp.bfloat16),
                    preferred_element_type=jnp.float32) + b1_ref[...]
        h = jnp.maximum(h, 0.0)
        o = jnp.dot(h.astype(jnp.bfloat16), w2_ref[...].astype(jnp.bfloat16),
                    preferred_element_type=jnp.float32) + b2_ref[...]
        if final_relu:
            o = jnp.maximum(o, 0.0)
        o_ref[...] = o

    return pl.pallas_call(
        body,
        grid=(nblk,),
        in_specs=[
            pl.BlockSpec((R, D), lambda i: (i, 0)),
            pl.BlockSpec((R, D), lambda i: (i, 0)),
            pl.BlockSpec((R, D), lambda i: (i + nblk, 0)),
            pl.BlockSpec((D, D), lambda i: (0, 0)),
            pl.BlockSpec((1, D), lambda i: (0, 0)),
            pl.BlockSpec((D, D), lambda i: (0, 0)),
            pl.BlockSpec((1, D), lambda i: (0, 0)),
        ],
        out_specs=pl.BlockSpec((R, D), lambda i: (i, 0)),
        out_shape=jax.ShapeDtypeStruct((N, D), jnp.float32),
    )(xin, partials, partials, W1, b1.reshape(1, D), W2, b2.reshape(1, D))


def kernel(x, edge_index, W1_0, b1_0, W2_0, b2_0, W1_1, b1_1, W2_1, b2_1):
    src = edge_index[0].astype(jnp.int32)
    dst = edge_index[1].astype(jnp.int32)
    pad = EPAD - E
    if pad:
        src_p = jnp.concatenate([src, jnp.zeros((pad,), jnp.int32)])
        dst_p = jnp.concatenate([dst, jnp.full((pad,), N, jnp.int32)])
    else:
        src_p, dst_p = src, dst
    p = _sc_scatter_partials(x, src_p, dst_p)
    h = _mlp(x, p, W1_0, b1_0, W2_0, b2_0, final_relu=True)
    p = _sc_scatter_partials(h, src_p, dst_p)
    return _mlp(h, p, W1_1, b1_1, W2_1, b2_1, final_relu=False)


# speedup vs baseline: 1.0155x; 1.0027x over previous
import functools

import jax
import jax.numpy as jnp
from jax import lax
from jax.experimental import pallas as pl
from jax.experimental.pallas import tpu as pltpu
from jax.experimental.pallas import tpu_sc as plsc

N = 10000
D = 128
E = 320000

NC = 2
NS = 16
CHUNK = 80
NCHUNK = 125
EPW = NCHUNK * CHUNK
EPAD = NC * NS * EPW
NIDX = 6
NRB = 3
NDUMMY = 16
ROW_STEP = 624


def _sc_scatter_partials(feat, src_p, dst_p):
    mesh = plsc.VectorSubcoreMesh(core_axis_name="c", subcore_axis_name="s")

    @functools.partial(
        pl.kernel,
        out_type=jax.ShapeDtypeStruct((NC * N, D), jnp.float32),
        mesh=mesh,
        scratch_types=[
            pltpu.VMEM_SHARED((N + NDUMMY, D), jnp.float32),
            pltpu.VMEM((NRB, CHUNK, D), jnp.float32),
            *([pltpu.VMEM((CHUNK,), jnp.int32)] * NIDX),
            *([pltpu.VMEM((CHUNK,), jnp.int32)] * NIDX),
            *([pltpu.SemaphoreType.DMA] * NIDX),
            *([pltpu.SemaphoreType.DMA] * NIDX),
            *([pltpu.SemaphoreType.DMA] * NRB),
            *([pltpu.SemaphoreType.DMA] * NRB),
        ],
    )
    def k(feat_hbm, src_hbm, dst_hbm, out_hbm, acc, rows, *rest):
        srci = list(rest[0:NIDX])
        dsti = list(rest[NIDX:2 * NIDX])
        ssem = list(rest[2 * NIDX:3 * NIDX])
        dsem = list(rest[3 * NIDX:4 * NIDX])
        gsem = list(rest[4 * NIDX:4 * NIDX + NRB])
        csem = list(rest[4 * NIDX + NRB:4 * NIDX + 2 * NRB])
        c = lax.axis_index("c")
        s = lax.axis_index("s")
        w = c * NS + s
        base = w * EPW
        row0 = s * ROW_STEP

        def idx_start(m, sl):
            off = base + m * CHUNK
            d1 = pltpu.async_copy(src_hbm.at[pl.ds(off, CHUNK)], srci[sl], ssem[sl])
            d2 = pltpu.async_copy(dst_hbm.at[pl.ds(off, CHUNK)], dsti[sl], dsem[sl])
            return (d1, d2)

        def gather_start(sl, b):
            return pltpu.async_copy(feat_hbm.at[srci[sl]], rows.at[b], gsem[b])

        idesc = [None] * NIDX
        for m0 in range(3):
            idesc[m0] = idx_start(m0, m0)

        zv = jnp.zeros((16,), jnp.float32)

        def zb(i, carry):
            rows[0, i // (D // 16), pl.ds((i % (D // 16)) * 16, 16)] = zv
            return carry

        lax.fori_loop(0, CHUNK * (D // 16), zb, 0)

        def init_to(nrows):
            for j in range(nrows // CHUNK):
                pltpu.sync_copy(rows.at[0], acc.at[pl.ds(row0 + j * CHUNK, CHUNK)])
            rem = nrows % CHUNK
            if rem:
                pltpu.sync_copy(rows.at[0].at[pl.ds(0, rem)],
                                acc.at[pl.ds(row0 + (nrows // CHUNK) * CHUNK, rem)])

        @pl.when(s == NS - 1)
        def _():
            init_to(640)

        @pl.when(s != NS - 1)
        def _():
            init_to(ROW_STEP)

        plsc.subcore_barrier()

        gdesc = [None] * NRB
        sdesc = [None] * NRB
        idesc[0][0].wait()
        gdesc[0] = gather_start(0, 0)
        for m in range(NCHUNK):
            b = m % NRB
            sl = m % NIDX
            if m + 3 < NCHUNK:
                idesc[(m + 3) % NIDX] = idx_start(m + 3, (m + 3) % NIDX)
            if m + 1 < NCHUNK:
                if m >= 2:
                    sdesc[(m + 1) % NRB].wait()
                idesc[(m + 1) % NIDX][0].wait()
                gdesc[(m + 1) % NRB] = gather_start((m + 1) % NIDX, (m + 1) % NRB)
            gdesc[b].wait()
            idesc[sl][1].wait()
            sdesc[b] = pltpu.async_copy(rows.at[b], acc.at[dsti[sl]], csem[b],
                                        add=True)
        for j in range(NCHUNK - 3, NCHUNK):
            sdesc[j % NRB].wait()

        plsc.subcore_barrier()

        @pl.when(s == NS - 1)
        def _():
            pltpu.sync_copy(acc.at[pl.ds(row0, 640)],
                            out_hbm.at[pl.ds(c * N + row0, 640)])

        @pl.when(s != NS - 1)
        def _():
            pltpu.sync_copy(acc.at[pl.ds(row0, ROW_STEP)],
                            out_hbm.at[pl.ds(c * N + row0, ROW_STEP)])

    return k(feat, src_p, dst_p)


def _mlp(xin, partials, W1, b1, W2, b2, final_relu):
    R = 5000
    nblk = N // R

    def body(x_ref, p0_ref, p1_ref, w1_ref, b1_ref, w2_ref, b2_ref, o_ref):
        h = x_ref[...] + p0_ref[...] + p1_ref[...]
        h = jnp.dot(h, w1_ref[...], preferred_element_type=jnp.float32) + b1_ref[...]
        h = jnp.maximum(h, 0.0)
        o = jnp.dot(h, w2_ref[...], preferred_element_type=jnp.float32) + b2_ref[...]
        if final_relu:
            o = jnp.maximum(o, 0.0)
        o_ref[...] = o

    return pl.pallas_call(
        body,
        grid=(nblk,),
        in_specs=[
            pl.BlockSpec((R, D), lambda i: (i, 0)),
            pl.BlockSpec((R, D), lambda i: (i, 0)),
            pl.BlockSpec((R, D), lambda i: (i + nblk, 0)),
            pl.BlockSpec((D, D), lambda i: (0, 0)),
            pl.BlockSpec((1, D), lambda i: (0, 0)),
            pl.BlockSpec((D, D), lambda i: (0, 0)),
            pl.BlockSpec((1, D), lambda i: (0, 0)),
        ],
        out_specs=pl.BlockSpec((R, D), lambda i: (i, 0)),
        out_shape=jax.ShapeDtypeStruct((N, D), jnp.float32),
    )(xin, partials, partials, W1, b1.reshape(1, D), W2, b2.reshape(1, D))


def kernel(x, edge_index, W1_0, b1_0, W2_0, b2_0, W1_1, b1_1, W2_1, b2_1):
    src = edge_index[0].astype(jnp.int32)
    dst = edge_index[1].astype(jnp.int32)
    pad = EPAD - E
    if pad:
        src_p = jnp.concatenate([src, jnp.zeros((pad,), jnp.int32)])
        dst_p = jnp.concatenate([dst, jnp.full((pad,), N, jnp.int32)])
    else:
        src_p, dst_p = src, dst
    p = _sc_scatter_partials(x, src_p, dst_p)
    h = _mlp(x, p, W1_0, b1_0, W2_0, b2_0, final_relu=True)
    p = _sc_scatter_partials(h, src_p, dst_p)
    return _mlp(h, p, W1_1, b1_1, W2_1, b2_1, final_relu=False)
